# baseline (speedup 1.0000x reference)
import functools

import jax
import jax.numpy as jnp
from jax import lax
from jax.experimental import pallas as pl
from jax.experimental.pallas import tpu as pltpu
from jax.experimental.pallas import tpu_sc as plsc

N = 10000
NREL = 200
E = 320000
D = 128
H = 4
ALPHA = 0.1
NITER = 4

NC = 2
NS = 16
LANES = 16
CH = 128
NCHUNK = E // CH
MAXC_SC = NCHUNK // NS + 1
NPT = N // NS
NPC = 125
BM = 1000

_F32 = jnp.float32
_I32 = jnp.int32


def _tc_node_body(x_ref, g_ref, b_ref, wh_ref, wt_ref, ath_ref, att_ref,
                  z0_ref, az0_ref, ahat_ref):
    x = x_ref[0]
    mu = jnp.mean(x, axis=-1, keepdims=True)
    var = jnp.mean((x - mu) ** 2, axis=-1, keepdims=True)
    hn = (x - mu) / jnp.sqrt(var + 1e-5) * g_ref[...] + b_ref[...]
    z0_ref[0, 0] = hn[:, :64]
    z0_ref[1, 0] = hn[:, 64:]
    az0_ref[0, 0] = ALPHA * hn[:, :64]
    az0_ref[1, 0] = ALPHA * hn[:, 64:]
    dn = (((1,), (1,)), ((), ()))
    th = jnp.tanh(lax.dot_general(hn, wh_ref[...], dn, preferred_element_type=_F32))
    tt = jnp.tanh(lax.dot_general(hn, wt_ref[...], dn, preferred_element_type=_F32))
    ah = jnp.sum(th.reshape(NPT, H, D) * ath_ref[...].reshape(1, H, D), axis=-1)
    at = jnp.sum(tt.reshape(NPT, H, D) * att_ref[...].reshape(1, H, D), axis=-1)
    ahat_ref[0] = jnp.concatenate([ah, at], axis=1)


def _tc_node(entity_r, gamma, beta, wh, wt, ath, att):
    return pl.pallas_call(
        _tc_node_body,
        grid=(NS,),
        in_specs=[
            pl.BlockSpec((1, NPT, D), lambda i: (i, 0, 0)),
            pl.BlockSpec((1, D), lambda i: (0, 0)),
            pl.BlockSpec((1, D), lambda i: (0, 0)),
            pl.BlockSpec((H * D, D), lambda i: (0, 0)),
            pl.BlockSpec((H * D, D), lambda i: (0, 0)),
            pl.BlockSpec((H, D), lambda i: (0, 0)),
            pl.BlockSpec((H, D), lambda i: (0, 0)),
        ],
        out_specs=[
            pl.BlockSpec((NC, 1, NPT, 64), lambda i: (0, i, 0, 0)),
            pl.BlockSpec((NC, 1, NPT, 64), lambda i: (0, i, 0, 0)),
            pl.BlockSpec((1, NPT, 2 * H), lambda i: (i, 0, 0)),
        ],
        out_shape=[
            jax.ShapeDtypeStruct((NC, NS, NPT, 64), _F32),
            jax.ShapeDtypeStruct((NC, NS, NPT, 64), _F32),
            jax.ShapeDtypeStruct((NS, NPT, 2 * H), _F32),
        ],
    )(entity_r, gamma.reshape(1, D), beta.reshape(1, D), wh, wt,
      ath.reshape(H, D), att.reshape(H, D))


def _tc_rel_body(r_ref, wr_ref, atr_ref, ar_ref):
    dn = (((1,), (1,)), ((), ()))
    tr = jnp.tanh(lax.dot_general(r_ref[...], wr_ref[...], dn,
                                  preferred_element_type=_F32))
    ar_ref[...] = jnp.sum(tr.reshape(NREL, H, D) * atr_ref[...].reshape(1, H, D),
                          axis=-1)


def _tc_rel(relation, wr, atr):
    return pl.pallas_call(
        _tc_rel_body,
        out_shape=jax.ShapeDtypeStruct((NREL, H), _F32),
    )(relation, wr, atr.reshape(H, D))


def _tc_den_body(d_ref, o_ref):
    o_ref[0] = d_ref[0, 0, :, 0:H] + d_ref[1, 0, :, 0:H]


def _tc_den(denoms_r):
    return pl.pallas_call(
        _tc_den_body,
        grid=(NS,),
        in_specs=[pl.BlockSpec((NC, 1, NPT, 16), lambda i: (0, i, 0, 0))],
        out_specs=pl.BlockSpec((1, NPT, H), lambda i: (i, 0, 0)),
        out_shape=jax.ShapeDtypeStruct((NS, NPT, H), _F32),
    )(denoms_r)


def _tc_out_body(z_ref, wo_ref, x_ref, o_ref):
    acc = x_ref[0]
    dn = (((1,), (1,)), ((), ()))
    for h in range(H):
        for cc in range(NC):
            zb = z_ref[h, cc, 0]
            w = wo_ref[:, h * D + cc * 64:h * D + (cc + 1) * 64]
            acc = acc + lax.dot_general(zb, w, dn, preferred_element_type=_F32)
    o_ref[0] = acc


def _tc_out(zout, wo, x_r):
    return pl.pallas_call(
        _tc_out_body,
        grid=(NS,),
        in_specs=[
            pl.BlockSpec((H, NC, 1, NPT, 64), lambda i: (0, 0, i, 0, 0)),
            pl.BlockSpec((D, H * D), lambda i: (0, 0)),
            pl.BlockSpec((1, NPT, D), lambda i: (i, 0, 0)),
        ],
        out_specs=pl.BlockSpec((1, NPT, D), lambda i: (i, 0, 0)),
        out_shape=jax.ShapeDtypeStruct((NS, NPT, D), _F32),
    )(zout, wo, x_r)


_MESH = dict(core_axis_name="c", subcore_axis_name="s")


def _full(v):
    return jnp.full((LANES,), v, _I32)


@functools.partial(
    pl.kernel,
    out_type=[
        jax.ShapeDtypeStruct((H * E,), _F32),
        jax.ShapeDtypeStruct((NC, N, 16), _F32),
    ],
    mesh=plsc.VectorSubcoreMesh(**_MESH),
    compiler_params=pltpu.CompilerParams(needs_layout_passes=False, use_tc_tiling_on_sc=False),
    scratch_types=(
        [pltpu.VMEM((N, 2 * H), _F32)]
        + [pltpu.VMEM((NREL, H), _F32)]
        + [pltpu.VMEM((CH,), _I32)] * 6
        + [pltpu.VMEM((CH, 16), _F32)] * 2
        + [pltpu.VMEM((H, CH), _F32)] * 2
        + [pltpu.SemaphoreType.DMA] * 6
        + [pltpu.VMEM_SHARED((N, 16), _F32)]
    ),
)
def _s1(ahat_hbm, ar_hbm, eh_hbm, et_hbm, ety_hbm, exh_hbm, den_hbm,
        aht, arv, hb0, hb1, tb0, tb1, yb0, yb1, x160, x161, xs0, xs1,
        le0, le1, we0, we1, ds0, ds1, den_sp):
    headb = (hb0, hb1)
    tailb = (tb0, tb1)
    typeb = (yb0, yb1)
    ex16 = (x160, x161)
    exst = (xs0, xs1)
    lsem = (le0, le1)
    wsem = (we0, we1)
    dsem = (ds0, ds1)
    c = lax.axis_index("c")
    s = lax.axis_index("s")
    w = c * NS + s
    pltpu.sync_copy(ahat_hbm, aht)
    pltpu.sync_copy(ar_hbm, arv)

    def _zro(i, carry):
        ex16[0][i, :] = jnp.zeros((LANES,), _F32)
        ex16[1][i, :] = jnp.zeros((LANES,), _F32)
        return carry
    lax.fori_loop(0, CH, _zro, 0)
    for j in range(NPT // NPC):
        pltpu.sync_copy(ex16[0].at[pl.ds(0, NPC)],
                        den_sp.at[pl.ds(s * NPT + j * NPC, NPC)])
    plsc.subcore_barrier()

    lo = (w * NCHUNK) // (NC * NS)
    hi = ((w + 1) * NCHUNK) // (NC * NS)
    nfull = (hi - lo) // 2

    def _compute(i):
        for h in range(H):
            for j in range(CH // LANES):
                sl = pl.ds(j * LANES, LANES)
                hv = headb[i][sl]
                tv = tailb[i][sl]
                rv = typeb[i][sl]
                a1 = plsc.load_gather(aht, [hv, _full(h)])
                a2 = plsc.load_gather(aht, [tv, _full(H + h)])
                a3 = plsc.load_gather(arv, [rv, _full(h)])
                sc = a1 + a2 + a3
                sc = jnp.where(sc > 0, sc, 0.01 * sc)
                ex = jnp.exp(sc)
                exst[i][h, sl] = ex
                rows = lax.iota(_I32, LANES) + j * LANES
                plsc.store_scatter(ex16[i], [rows, _full(h)], ex)

    def _batch(kk, carry):
        ld = []
        for i in range(2):
            off = (lo + kk * 2 + i) * CH
            ld.append((
                pltpu.async_copy(eh_hbm.at[pl.ds(off, CH)], headb[i], lsem[i]),
                pltpu.async_copy(et_hbm.at[pl.ds(off, CH)], tailb[i], lsem[i]),
                pltpu.async_copy(ety_hbm.at[pl.ds(off, CH)], typeb[i], lsem[i]),
            ))
        wd = []
        for i in range(2):
            off = (lo + kk * 2 + i) * CH
            for dsc in ld[i]:
                dsc.wait()
            _compute(i)
            for h in range(H):
                wd.append(pltpu.async_copy(exst[i].at[h],
                                           exh_hbm.at[pl.ds(h * E + off, CH)],
                                           wsem[i]))
            wd.append(pltpu.async_copy(ex16[i], den_sp.at[headb[i]], dsem[i],
                                       add=True))
        for dsc in wd:
            dsc.wait()
        return carry
    lax.fori_loop(0, nfull, _batch, 0)

    def _chunk(k, carry):
        off = k * CH
        pltpu.sync_copy(eh_hbm.at[pl.ds(off, CH)], headb[0])
        pltpu.sync_copy(et_hbm.at[pl.ds(off, CH)], tailb[0])
        pltpu.sync_copy(ety_hbm.at[pl.ds(off, CH)], typeb[0])
        _compute(0)
        for h in range(H):
            pltpu.sync_copy(exst[0].at[h], exh_hbm.at[pl.ds(h * E + off, CH)])
        pltpu.sync_copy(ex16[0], den_sp.at[headb[0]], add=True)
        return carry
    lax.fori_loop(lo + nfull * 2, hi, _chunk, 0)
    plsc.subcore_barrier()

    @pl.when(s == 0)
    def _():
        pltpu.sync_copy(den_sp, den_hbm.at[c])


@functools.partial(
    pl.kernel,
    out_type=jax.ShapeDtypeStruct((H * E,), _F32),
    mesh=plsc.VectorSubcoreMesh(**_MESH),
    compiler_params=pltpu.CompilerParams(needs_layout_passes=False, use_tc_tiling_on_sc=False),
    scratch_types=(
        [pltpu.VMEM((N, H), _F32)]
        + [pltpu.VMEM((CH,), _I32)] * 2
        + [pltpu.VMEM((H, CH), _F32)] * 2
        + [pltpu.VMEM((H, CH), _F32)] * 2
        + [pltpu.SemaphoreType.DMA] * 4
    ),
)
def _s2(exh_hbm, den4_hbm, eh_hbm, a_hbm, den4, hb0, hb1, exb0, exb1,
        ast0, ast1, le0, le1, wr0, wr1):
    headb = (hb0, hb1)
    exb = (exb0, exb1)
    ast = (ast0, ast1)
    lsem = (le0, le1)
    wsem = (wr0, wr1)
    c = lax.axis_index("c")
    s = lax.axis_index("s")
    w = c * NS + s
    pltpu.sync_copy(den4_hbm, den4)
    lo = (w * NCHUNK) // (NC * NS)
    hi = ((w + 1) * NCHUNK) // (NC * NS)
    nfull = (hi - lo) // 2

    def _compute(i, off):
        for h in range(H):
            for j in range(CH // LANES):
                sl = pl.ds(j * LANES, LANES)
                hv = headb[i][sl]
                dv = plsc.load_gather(den4, [hv, _full(h)])
                ast[i][h, sl] = (1.0 - ALPHA) * exb[i][h, sl] / (dv + 1e-30)

    def _batch(kk, carry):
        ld = []
        for i in range(2):
            off = (lo + kk * 2 + i) * CH
            d = [pltpu.async_copy(eh_hbm.at[pl.ds(off, CH)], headb[i], lsem[i])]
            for h in range(H):
                d.append(pltpu.async_copy(exh_hbm.at[pl.ds(h * E + off, CH)],
                                          exb[i].at[h], lsem[i]))
            ld.append(d)
        wd = []
        for i in range(2):
            off = (lo + kk * 2 + i) * CH
            for dsc in ld[i]:
                dsc.wait()
            _compute(i, off)
            for h in range(H):
                wd.append(pltpu.async_copy(ast[i].at[h],
                                           a_hbm.at[pl.ds(h * E + off, CH)],
                                           wsem[i]))
        for dsc in wd:
            dsc.wait()
        return carry
    lax.fori_loop(0, nfull, _batch, 0)

    def _chunk(k, carry):
        off = k * CH
        pltpu.sync_copy(eh_hbm.at[pl.ds(off, CH)], headb[0])
        for h in range(H):
            pltpu.sync_copy(exh_hbm.at[pl.ds(h * E + off, CH)], exb[0].at[h])
        _compute(0, off)
        for h in range(H):
            pltpu.sync_copy(ast[0].at[h], a_hbm.at[pl.ds(h * E + off, CH)])
        return carry
    lax.fori_loop(lo + nfull * 2, hi, _chunk, 0)


NB = 5


@functools.partial(
    pl.kernel,
    out_type=jax.ShapeDtypeStruct((H, NC, NS, NPT, 64), _F32),
    mesh=plsc.VectorSubcoreMesh(**_MESH),
    compiler_params=pltpu.CompilerParams(needs_layout_passes=False, use_tc_tiling_on_sc=False),
    scratch_types=(
        [pltpu.VMEM_SHARED((N, 64), _F32)] * 2
        + [pltpu.VMEM((CH,), _I32)] * NB
        + [pltpu.VMEM((CH,), _I32)] * NB
        + [pltpu.VMEM((CH,), _F32)] * NB
        + [pltpu.VMEM((CH, 64), _F32)] * NB
        + [pltpu.SemaphoreType.DMA] * (3 * NB)
    ),
)
def _s3(a_hbm, z0_hbm, az0_hbm, et_hbm, eh_hbm, zout_hbm, *refs):
    za_sp, zb_sp = refs[0], refs[1]
    tail = refs[2:2 + NB]
    head = refs[2 + NB:2 + 2 * NB]
    ac = refs[2 + 2 * NB:2 + 3 * NB]
    rows = refs[2 + 3 * NB:2 + 4 * NB]
    esem = refs[2 + 4 * NB:2 + 5 * NB]
    gsem = refs[2 + 5 * NB:2 + 6 * NB]
    ssem = refs[2 + 6 * NB:2 + 7 * NB]
    c = lax.axis_index("c")
    s = lax.axis_index("s")
    klo = (s * NCHUNK) // NS
    khi = ((s + 1) * NCHUNK) // NS
    nfull = (khi - klo) // NB
    nb = s * NPT

    def _mul(rows_i, ac_i):
        def _m(r):
            wv = plsc.load_gather(ac_i, [_full(0) + r])
            for q in range(4):
                sl = pl.ds(q * LANES, LANES)
                rows_i[r, sl] = rows_i[r, sl] * wv
        plsc.parallel_loop(0, CH, 1, unroll=16)(_m)

    def _head(h, carry):
        pltpu.sync_copy(z0_hbm.at[c, s], za_sp.at[pl.ds(nb, NPT)])
        cur = za_sp
        for it in range(NITER):
            nxt = (zb_sp, za_sp)[it % 2]
            pltpu.sync_copy(az0_hbm.at[c, s], nxt.at[pl.ds(nb, NPT)])
            plsc.subcore_barrier()

            def _batch(kk, carry2):
                kb = klo + kk * NB
                ed = []
                for i in range(NB):
                    off = (kb + i) * CH
                    ed.append((
                        pltpu.async_copy(et_hbm.at[pl.ds(off, CH)], tail[i], esem[i]),
                        pltpu.async_copy(eh_hbm.at[pl.ds(off, CH)], head[i], esem[i]),
                        pltpu.async_copy(a_hbm.at[pl.ds(h * E + off, CH)], ac[i], esem[i]),
                    ))
                gd = []
                for i in range(NB):
                    for dsc in ed[i]:
                        dsc.wait()
                    gd.append(pltpu.async_copy(cur.at[tail[i]], rows[i], gsem[i]))
                sd = []
                for i in range(NB):
                    gd[i].wait()
                    _mul(rows[i], ac[i])
                    sd.append(pltpu.async_copy(rows[i], nxt.at[head[i]], ssem[i],
                                               add=True))
                for dsc in sd:
                    dsc.wait()
                return carry2
            lax.fori_loop(0, nfull, _batch, 0)

            def _edge(k, carry2):
                off = k * CH
                pltpu.sync_copy(et_hbm.at[pl.ds(off, CH)], tail[0])
                pltpu.sync_copy(eh_hbm.at[pl.ds(off, CH)], head[0])
                pltpu.sync_copy(a_hbm.at[pl.ds(h * E + off, CH)], ac[0])
                pltpu.async_copy(cur.at[tail[0]], rows[0], gsem[0]).wait()
                _mul(rows[0], ac[0])
                pltpu.sync_copy(rows[0], nxt.at[head[0]], add=True)
                return carry2
            lax.fori_loop(klo + nfull * NB, khi, _edge, 0)
            plsc.subcore_barrier()
            cur = nxt
        pltpu.sync_copy(cur.at[pl.ds(nb, NPT)], zout_hbm.at[h, c, s])
        return carry
    lax.fori_loop(0, H, _head, 0)


def kernel(params, edge_index, edge_type):
    entity = params['entity_embed']
    relation = params['relation_embed']
    e_head = edge_index[0]
    e_tail = edge_index[1]
    for lp in params['layers']:
        z0, az0, ahat = _tc_node(entity.reshape(NS, NPT, D),
                                 lp['norm_gamma'], lp['norm_beta'],
                                 lp['W_h'], lp['W_t'], lp['att_h'], lp['att_t'])
        ar = _tc_rel(relation, lp['W_r'], lp['att_r'])
        exh, denoms = _s1(ahat.reshape(N, 2 * H), ar, e_head, e_tail, edge_type)
        den4 = _tc_den(denoms.reshape(NC, NS, NPT, 16))
        a = _s2(exh, den4.reshape(N, H), e_head)
        zout = _s3(a, z0, az0, e_tail, e_head)
        entity = _tc_out(zout, lp['W_o'],
                         entity.reshape(NS, NPT, D)).reshape(N, D)
    return entity


# speedup vs baseline: 1.0102x; 1.0102x over previous
import functools

import jax
import jax.numpy as jnp
from jax import lax
from jax.experimental import pallas as pl
from jax.experimental.pallas import tpu as pltpu
from jax.experimental.pallas import tpu_sc as plsc

N = 10000
NREL = 200
E = 320000
D = 128
H = 4
ALPHA = 0.1
NITER = 4

NC = 2
NS = 16
LANES = 16
CH = 128
NCHUNK = E // CH
MAXC_SC = NCHUNK // NS + 1
NPT = N // NS
NPC = 125
BM = 1000

_F32 = jnp.float32
_I32 = jnp.int32


def _tc_node_body(x_ref, g_ref, b_ref, wh_ref, wt_ref, ath_ref, att_ref,
                  z0_ref, az0_ref, ahat_ref):
    x = x_ref[0]
    mu = jnp.mean(x, axis=-1, keepdims=True)
    var = jnp.mean((x - mu) ** 2, axis=-1, keepdims=True)
    hn = (x - mu) / jnp.sqrt(var + 1e-5) * g_ref[...] + b_ref[...]
    z0_ref[0, 0] = hn[:, :64]
    z0_ref[1, 0] = hn[:, 64:]
    az0_ref[0, 0] = ALPHA * hn[:, :64]
    az0_ref[1, 0] = ALPHA * hn[:, 64:]
    dn = (((1,), (1,)), ((), ()))
    th = jnp.tanh(lax.dot_general(hn, wh_ref[...], dn, preferred_element_type=_F32))
    tt = jnp.tanh(lax.dot_general(hn, wt_ref[...], dn, preferred_element_type=_F32))
    ah = jnp.sum(th.reshape(NPT, H, D) * ath_ref[...].reshape(1, H, D), axis=-1)
    at = jnp.sum(tt.reshape(NPT, H, D) * att_ref[...].reshape(1, H, D), axis=-1)
    ahat_ref[0] = jnp.concatenate([ah, at], axis=1)


def _tc_node(entity_r, gamma, beta, wh, wt, ath, att):
    return pl.pallas_call(
        _tc_node_body,
        grid=(NS,),
        in_specs=[
            pl.BlockSpec((1, NPT, D), lambda i: (i, 0, 0)),
            pl.BlockSpec((1, D), lambda i: (0, 0)),
            pl.BlockSpec((1, D), lambda i: (0, 0)),
            pl.BlockSpec((H * D, D), lambda i: (0, 0)),
            pl.BlockSpec((H * D, D), lambda i: (0, 0)),
            pl.BlockSpec((H, D), lambda i: (0, 0)),
            pl.BlockSpec((H, D), lambda i: (0, 0)),
        ],
        out_specs=[
            pl.BlockSpec((NC, 1, NPT, 64), lambda i: (0, i, 0, 0)),
            pl.BlockSpec((NC, 1, NPT, 64), lambda i: (0, i, 0, 0)),
            pl.BlockSpec((1, NPT, 2 * H), lambda i: (i, 0, 0)),
        ],
        out_shape=[
            jax.ShapeDtypeStruct((NC, NS, NPT, 64), _F32),
            jax.ShapeDtypeStruct((NC, NS, NPT, 64), _F32),
            jax.ShapeDtypeStruct((NS, NPT, 2 * H), _F32),
        ],
    )(entity_r, gamma.reshape(1, D), beta.reshape(1, D), wh, wt,
      ath.reshape(H, D), att.reshape(H, D))


def _tc_rel_body(r_ref, wr_ref, atr_ref, ar_ref):
    dn = (((1,), (1,)), ((), ()))
    tr = jnp.tanh(lax.dot_general(r_ref[...], wr_ref[...], dn,
                                  preferred_element_type=_F32))
    ar_ref[...] = jnp.sum(tr.reshape(NREL, H, D) * atr_ref[...].reshape(1, H, D),
                          axis=-1)


def _tc_rel(relation, wr, atr):
    return pl.pallas_call(
        _tc_rel_body,
        out_shape=jax.ShapeDtypeStruct((NREL, H), _F32),
    )(relation, wr, atr.reshape(H, D))


def _tc_den_body(d_ref, o_ref):
    o_ref[0] = d_ref[0, 0, :, 0:H] + d_ref[1, 0, :, 0:H]


def _tc_den(denoms_r):
    return pl.pallas_call(
        _tc_den_body,
        grid=(NS,),
        in_specs=[pl.BlockSpec((NC, 1, NPT, 16), lambda i: (0, i, 0, 0))],
        out_specs=pl.BlockSpec((1, NPT, H), lambda i: (i, 0, 0)),
        out_shape=jax.ShapeDtypeStruct((NS, NPT, H), _F32),
    )(denoms_r)


def _tc_out_body(z_ref, wo_ref, x_ref, o_ref):
    acc = x_ref[0]
    dn = (((1,), (1,)), ((), ()))
    for h in range(H):
        for cc in range(NC):
            zb = z_ref[h, cc, 0]
            w = wo_ref[:, h * D + cc * 64:h * D + (cc + 1) * 64]
            acc = acc + lax.dot_general(zb, w, dn, preferred_element_type=_F32)
    o_ref[0] = acc


def _tc_out(zout, wo, x_r):
    return pl.pallas_call(
        _tc_out_body,
        grid=(NS,),
        in_specs=[
            pl.BlockSpec((H, NC, 1, NPT, 64), lambda i: (0, 0, i, 0, 0)),
            pl.BlockSpec((D, H * D), lambda i: (0, 0)),
            pl.BlockSpec((1, NPT, D), lambda i: (i, 0, 0)),
        ],
        out_specs=pl.BlockSpec((1, NPT, D), lambda i: (i, 0, 0)),
        out_shape=jax.ShapeDtypeStruct((NS, NPT, D), _F32),
    )(zout, wo, x_r)


_MESH = dict(core_axis_name="c", subcore_axis_name="s")


def _full(v):
    return jnp.full((LANES,), v, _I32)


@functools.partial(
    pl.kernel,
    out_type=[
        jax.ShapeDtypeStruct((H * E,), _F32),
        jax.ShapeDtypeStruct((NC, N, 16), _F32),
    ],
    mesh=plsc.VectorSubcoreMesh(**_MESH),
    compiler_params=pltpu.CompilerParams(needs_layout_passes=False, use_tc_tiling_on_sc=False),
    scratch_types=(
        [pltpu.VMEM((N, 2 * H), _F32)]
        + [pltpu.VMEM((NREL, H), _F32)]
        + [pltpu.VMEM((CH,), _I32)] * 6
        + [pltpu.VMEM((CH, 16), _F32)] * 2
        + [pltpu.VMEM((H, CH), _F32)] * 2
        + [pltpu.SemaphoreType.DMA] * 6
        + [pltpu.VMEM_SHARED((N, 16), _F32)]
    ),
)
def _s1(ahat_hbm, ar_hbm, eh_hbm, et_hbm, ety_hbm, exh_hbm, den_hbm,
        aht, arv, hb0, hb1, tb0, tb1, yb0, yb1, x160, x161, xs0, xs1,
        le0, le1, we0, we1, ds0, ds1, den_sp):
    headb = (hb0, hb1)
    tailb = (tb0, tb1)
    typeb = (yb0, yb1)
    ex16 = (x160, x161)
    exst = (xs0, xs1)
    lsem = (le0, le1)
    wsem = (we0, we1)
    dsem = (ds0, ds1)
    c = lax.axis_index("c")
    s = lax.axis_index("s")
    w = c * NS + s
    pltpu.sync_copy(ahat_hbm, aht)
    pltpu.sync_copy(ar_hbm, arv)

    def _zro(i, carry):
        ex16[0][i, :] = jnp.zeros((LANES,), _F32)
        ex16[1][i, :] = jnp.zeros((LANES,), _F32)
        return carry
    lax.fori_loop(0, CH, _zro, 0)
    for j in range(NPT // NPC):
        pltpu.sync_copy(ex16[0].at[pl.ds(0, NPC)],
                        den_sp.at[pl.ds(s * NPT + j * NPC, NPC)])
    plsc.subcore_barrier()

    lo = (w * NCHUNK) // (NC * NS)
    hi = ((w + 1) * NCHUNK) // (NC * NS)
    nfull = (hi - lo) // 2

    def _compute(i):
        for h in range(H):
            for j in range(CH // LANES):
                sl = pl.ds(j * LANES, LANES)
                hv = headb[i][sl]
                tv = tailb[i][sl]
                rv = typeb[i][sl]
                a1 = plsc.load_gather(aht, [hv, _full(h)])
                a2 = plsc.load_gather(aht, [tv, _full(H + h)])
                a3 = plsc.load_gather(arv, [rv, _full(h)])
                sc = a1 + a2 + a3
                sc = jnp.where(sc > 0, sc, 0.01 * sc)
                ex = jnp.exp(sc)
                exst[i][h, sl] = ex
                rows = lax.iota(_I32, LANES) + j * LANES
                plsc.store_scatter(ex16[i], [rows, _full(h)], ex)

    def _batch(kk, carry):
        ld = []
        for i in range(2):
            off = (lo + kk * 2 + i) * CH
            ld.append((
                pltpu.async_copy(eh_hbm.at[pl.ds(off, CH)], headb[i], lsem[i]),
                pltpu.async_copy(et_hbm.at[pl.ds(off, CH)], tailb[i], lsem[i]),
                pltpu.async_copy(ety_hbm.at[pl.ds(off, CH)], typeb[i], lsem[i]),
            ))
        wd = []
        for i in range(2):
            off = (lo + kk * 2 + i) * CH
            for dsc in ld[i]:
                dsc.wait()
            _compute(i)
            for h in range(H):
                wd.append(pltpu.async_copy(exst[i].at[h],
                                           exh_hbm.at[pl.ds(h * E + off, CH)],
                                           wsem[i]))
            wd.append(pltpu.async_copy(ex16[i], den_sp.at[headb[i]], dsem[i],
                                       add=True))
        for dsc in wd:
            dsc.wait()
        return carry
    lax.fori_loop(0, nfull, _batch, 0)

    def _chunk(k, carry):
        off = k * CH
        pltpu.sync_copy(eh_hbm.at[pl.ds(off, CH)], headb[0])
        pltpu.sync_copy(et_hbm.at[pl.ds(off, CH)], tailb[0])
        pltpu.sync_copy(ety_hbm.at[pl.ds(off, CH)], typeb[0])
        _compute(0)
        for h in range(H):
            pltpu.sync_copy(exst[0].at[h], exh_hbm.at[pl.ds(h * E + off, CH)])
        pltpu.sync_copy(ex16[0], den_sp.at[headb[0]], add=True)
        return carry
    lax.fori_loop(lo + nfull * 2, hi, _chunk, 0)
    plsc.subcore_barrier()

    @pl.when(s == 0)
    def _():
        pltpu.sync_copy(den_sp, den_hbm.at[c])


@functools.partial(
    pl.kernel,
    out_type=jax.ShapeDtypeStruct((H * E,), _F32),
    mesh=plsc.VectorSubcoreMesh(**_MESH),
    compiler_params=pltpu.CompilerParams(needs_layout_passes=False, use_tc_tiling_on_sc=False),
    scratch_types=(
        [pltpu.VMEM((N, H), _F32)]
        + [pltpu.VMEM((CH,), _I32)] * 2
        + [pltpu.VMEM((H, CH), _F32)] * 2
        + [pltpu.VMEM((H, CH), _F32)] * 2
        + [pltpu.SemaphoreType.DMA] * 4
    ),
)
def _s2(exh_hbm, den4_hbm, eh_hbm, a_hbm, den4, hb0, hb1, exb0, exb1,
        ast0, ast1, le0, le1, wr0, wr1):
    headb = (hb0, hb1)
    exb = (exb0, exb1)
    ast = (ast0, ast1)
    lsem = (le0, le1)
    wsem = (wr0, wr1)
    c = lax.axis_index("c")
    s = lax.axis_index("s")
    w = c * NS + s
    pltpu.sync_copy(den4_hbm, den4)
    lo = (w * NCHUNK) // (NC * NS)
    hi = ((w + 1) * NCHUNK) // (NC * NS)
    nfull = (hi - lo) // 2

    def _compute(i, off):
        for h in range(H):
            for j in range(CH // LANES):
                sl = pl.ds(j * LANES, LANES)
                hv = headb[i][sl]
                dv = plsc.load_gather(den4, [hv, _full(h)])
                ast[i][h, sl] = (1.0 - ALPHA) * exb[i][h, sl] / (dv + 1e-30)

    def _batch(kk, carry):
        ld = []
        for i in range(2):
            off = (lo + kk * 2 + i) * CH
            d = [pltpu.async_copy(eh_hbm.at[pl.ds(off, CH)], headb[i], lsem[i])]
            for h in range(H):
                d.append(pltpu.async_copy(exh_hbm.at[pl.ds(h * E + off, CH)],
                                          exb[i].at[h], lsem[i]))
            ld.append(d)
        wd = []
        for i in range(2):
            off = (lo + kk * 2 + i) * CH
            for dsc in ld[i]:
                dsc.wait()
            _compute(i, off)
            for h in range(H):
                wd.append(pltpu.async_copy(ast[i].at[h],
                                           a_hbm.at[pl.ds(h * E + off, CH)],
                                           wsem[i]))
        for dsc in wd:
            dsc.wait()
        return carry
    lax.fori_loop(0, nfull, _batch, 0)

    def _chunk(k, carry):
        off = k * CH
        pltpu.sync_copy(eh_hbm.at[pl.ds(off, CH)], headb[0])
        for h in range(H):
            pltpu.sync_copy(exh_hbm.at[pl.ds(h * E + off, CH)], exb[0].at[h])
        _compute(0, off)
        for h in range(H):
            pltpu.sync_copy(ast[0].at[h], a_hbm.at[pl.ds(h * E + off, CH)])
        return carry
    lax.fori_loop(lo + nfull * 2, hi, _chunk, 0)


NB = 5


@functools.partial(
    pl.kernel,
    out_type=jax.ShapeDtypeStruct((H, NC, NS, NPT, 64), _F32),
    mesh=plsc.VectorSubcoreMesh(**_MESH),
    compiler_params=pltpu.CompilerParams(needs_layout_passes=False, use_tc_tiling_on_sc=False),
    scratch_types=(
        [pltpu.VMEM_SHARED((N, 64), _F32)] * 2
        + [pltpu.VMEM((CH,), _I32)] * NB
        + [pltpu.VMEM((CH,), _I32)] * NB
        + [pltpu.VMEM((CH,), _F32)] * NB
        + [pltpu.VMEM((CH, 64), _F32)] * NB
        + [pltpu.SemaphoreType.DMA] * (3 * NB)
    ),
)
def _s3(a_hbm, z0_hbm, az0_hbm, et_hbm, eh_hbm, zout_hbm, *refs):
    za_sp, zb_sp = refs[0], refs[1]
    tail = refs[2:2 + NB]
    head = refs[2 + NB:2 + 2 * NB]
    ac = refs[2 + 2 * NB:2 + 3 * NB]
    rows = refs[2 + 3 * NB:2 + 4 * NB]
    esem = refs[2 + 4 * NB:2 + 5 * NB]
    gsem = refs[2 + 5 * NB:2 + 6 * NB]
    ssem = refs[2 + 6 * NB:2 + 7 * NB]
    c = lax.axis_index("c")
    s = lax.axis_index("s")
    klo = (s * NCHUNK) // NS
    khi = ((s + 1) * NCHUNK) // NS
    nfull = (khi - klo) // NB
    nb = s * NPT

    def _mul(rows_i, ac_i):
        def _m(r):
            wv = plsc.load_gather(ac_i, [_full(0) + r])
            for q in range(4):
                sl = pl.ds(q * LANES, LANES)
                rows_i[r, sl] = rows_i[r, sl] * wv
        plsc.parallel_loop(0, CH, 1, unroll=8)(_m)

    def _head(h, carry):
        pltpu.sync_copy(z0_hbm.at[c, s], za_sp.at[pl.ds(nb, NPT)])
        cur = za_sp
        for it in range(NITER):
            nxt = (zb_sp, za_sp)[it % 2]
            pltpu.sync_copy(az0_hbm.at[c, s], nxt.at[pl.ds(nb, NPT)])
            plsc.subcore_barrier()

            def _batch(kk, carry2):
                kb = klo + kk * NB
                ed = []
                for i in range(NB):
                    off = (kb + i) * CH
                    ed.append((
                        pltpu.async_copy(et_hbm.at[pl.ds(off, CH)], tail[i], esem[i]),
                        pltpu.async_copy(eh_hbm.at[pl.ds(off, CH)], head[i], esem[i]),
                        pltpu.async_copy(a_hbm.at[pl.ds(h * E + off, CH)], ac[i], esem[i]),
                    ))
                gd = []
                for i in range(NB):
                    for dsc in ed[i]:
                        dsc.wait()
                    gd.append(pltpu.async_copy(cur.at[tail[i]], rows[i], gsem[i]))
                sd = []
                for i in range(NB):
                    gd[i].wait()
                    _mul(rows[i], ac[i])
                    sd.append(pltpu.async_copy(rows[i], nxt.at[head[i]], ssem[i],
                                               add=True))
                for dsc in sd:
                    dsc.wait()
                return carry2
            lax.fori_loop(0, nfull, _batch, 0)

            def _edge(k, carry2):
                off = k * CH
                pltpu.sync_copy(et_hbm.at[pl.ds(off, CH)], tail[0])
                pltpu.sync_copy(eh_hbm.at[pl.ds(off, CH)], head[0])
                pltpu.sync_copy(a_hbm.at[pl.ds(h * E + off, CH)], ac[0])
                pltpu.async_copy(cur.at[tail[0]], rows[0], gsem[0]).wait()
                _mul(rows[0], ac[0])
                pltpu.sync_copy(rows[0], nxt.at[head[0]], add=True)
                return carry2
            lax.fori_loop(klo + nfull * NB, khi, _edge, 0)
            plsc.subcore_barrier()
            cur = nxt
        pltpu.sync_copy(cur.at[pl.ds(nb, NPT)], zout_hbm.at[h, c, s])
        return carry
    lax.fori_loop(0, H, _head, 0)


def kernel(params, edge_index, edge_type):
    entity = params['entity_embed']
    relation = params['relation_embed']
    e_head = edge_index[0]
    e_tail = edge_index[1]
    for lp in params['layers']:
        z0, az0, ahat = _tc_node(entity.reshape(NS, NPT, D),
                                 lp['norm_gamma'], lp['norm_beta'],
                                 lp['W_h'], lp['W_t'], lp['att_h'], lp['att_t'])
        ar = _tc_rel(relation, lp['W_r'], lp['att_r'])
        exh, denoms = _s1(ahat.reshape(N, 2 * H), ar, e_head, e_tail, edge_type)
        den4 = _tc_den(denoms.reshape(NC, NS, NPT, 16))
        a = _s2(exh, den4.reshape(N, H), e_head)
        zout = _s3(a, z0, az0, e_tail, e_head)
        entity = _tc_out(zout, lp['W_o'],
                         entity.reshape(NS, NPT, D)).reshape(N, D)
    return entity
